# baseline (speedup 1.0000x reference)
import jax
import jax.numpy as jnp
from jax import lax
from jax.experimental import pallas as pl
from jax.experimental.pallas import tpu as pltpu
from jax.experimental.pallas import tpu_sc as plsc

NUM_CORES = 2
NUM_SUBCORES = 16
NUM_WORKERS = NUM_CORES * NUM_SUBCORES
G = 128
NBUF = 8
EMB = 64


def _body(idx_hbm, table_hbm, out_hbm, idx_v, rows, gsems, wsems):
    c = lax.axis_index("c")
    s = lax.axis_index("s")
    wid = s * NUM_CORES + c
    n_grp = idx_hbm.shape[0] // NUM_WORKERS
    base = wid * n_grp
    pltpu.sync_copy(idx_hbm.at[pl.ds(base, n_grp)], idx_v)

    def fire_gather_dyn(j, b):
        pltpu.async_copy(table_hbm.at[idx_v.at[j]], rows[b], gsems[b])

    def fire_write_dyn(j, b):
        pltpu.async_copy(
            rows[b], out_hbm.at[pl.ds((base + j) * G, G)], wsems[b]
        )

    def wait_g(b):
        pltpu.make_async_copy(table_hbm.at[pl.ds(0, G)], rows[b], gsems[b]).wait()

    def wait_w(b):
        pltpu.make_async_copy(rows[b], out_hbm.at[pl.ds(0, G)], wsems[b]).wait()

    for j in range(NBUF):
        fire_gather_dyn(j, j % NBUF)
        if j >= NBUF - 1:
            jj = j - (NBUF - 1)
            wait_g(jj % NBUF)
            fire_write_dyn(jj, jj % NBUF)

    def round_body(r, carry):
        j0 = NBUF + r * NBUF
        for b in range(NBUF):
            j = j0 + b
            wait_w(b)
            fire_gather_dyn(j, b)
            bb = (b + 1) % NBUF
            wait_g(bb)
            fire_write_dyn(j - (NBUF - 1), bb)
        return carry

    n_rounds = (n_grp - NBUF) // NBUF
    lax.fori_loop(0, n_rounds, round_body, 0)

    for j in range(n_grp, n_grp + NBUF - 1):
        jj = j - (NBUF - 1)
        wait_g(jj % NBUF)
        fire_write_dyn(jj, jj % NBUF)
    for b in range(NBUF):
        wait_w(b)


@jax.jit
def kernel(token_ids, weights):
    B, S = token_ids.shape
    n = B * S
    n_grp_total = n // G
    idx = token_ids.reshape(n_grp_total, G).astype(jnp.int32)
    mesh = plsc.VectorSubcoreMesh(core_axis_name="c", subcore_axis_name="s")
    n_grp = n_grp_total // NUM_WORKERS
    out = pl.kernel(
        _body,
        out_type=jax.ShapeDtypeStruct((n, EMB), jnp.float32),
        mesh=mesh,
        scratch_types=[
            pltpu.VMEM((n_grp, G), jnp.int32),
            [pltpu.VMEM((G, EMB), jnp.float32) for _ in range(NBUF)],
            [pltpu.SemaphoreType.DMA for _ in range(NBUF)],
            [pltpu.SemaphoreType.DMA for _ in range(NBUF)],
        ],
        compiler_params=pltpu.CompilerParams(use_tc_tiling_on_sc=False),
    )(idx, weights)
    return out.reshape(B, S, EMB)


# speedup vs baseline: 1.2331x; 1.2331x over previous
import jax
import jax.numpy as jnp
from jax import lax
from jax.experimental import pallas as pl
from jax.experimental.pallas import tpu as pltpu
from jax.experimental.pallas import tpu_sc as plsc
from jax.experimental.layout import Layout, with_layout_constraint

NUM_CORES = 2
NUM_SUBCORES = 16
NUM_WORKERS = NUM_CORES * NUM_SUBCORES
G = 128
NBUF = 4
EMB = 64
PADDED = 128
SUBPAD = 56


def _body(idx_hbm, drow_hbm, table_hbm, out_hbm, idx_v, drow_v, rows, gsems, wsems):
    c = lax.axis_index("c")
    s = lax.axis_index("s")
    wid = s * NUM_CORES + c
    n_grp = idx_hbm.shape[0] // NUM_WORKERS
    base = wid * n_grp
    pltpu.sync_copy(idx_hbm.at[pl.ds(base, n_grp)], idx_v)
    pltpu.sync_copy(drow_hbm.at[pl.ds(base, n_grp)], drow_v)

    def fire_gather(j, b):
        pltpu.async_copy(table_hbm.at[idx_v.at[j]], rows[b], gsems[b])

    def fire_write(j, b):
        pltpu.async_copy(rows[b], out_hbm.at[drow_v.at[j]], wsems[b])

    def wait_g(b):
        pltpu.make_async_copy(table_hbm.at[pl.ds(0, G)], rows[b], gsems[b]).wait()

    def wait_w(b):
        pltpu.make_async_copy(rows[b], out_hbm.at[pl.ds(0, G)], wsems[b]).wait()

    for j in range(NBUF):
        fire_gather(j, j % NBUF)
        if j >= NBUF - 1:
            jj = j - (NBUF - 1)
            wait_g(jj % NBUF)
            fire_write(jj, jj % NBUF)

    def round_body(r, carry):
        j0 = NBUF + r * NBUF
        for b in range(NBUF):
            j = j0 + b
            wait_w(b)
            fire_gather(j, b)
            bb = (b + 1) % NBUF
            wait_g(bb)
            fire_write(j - (NBUF - 1), bb)
        return carry

    n_rounds = (n_grp - NBUF) // NBUF
    lax.fori_loop(0, n_rounds, round_body, 0)

    for j in range(n_grp, n_grp + NBUF - 1):
        jj = j - (NBUF - 1)
        wait_g(jj % NBUF)
        fire_write(jj, jj % NBUF)
    for b in range(NBUF):
        wait_w(b)


@jax.jit
def kernel(token_ids, weights):
    B, S = token_ids.shape
    n = B * S
    n_grp_total = n // G
    idx = token_ids.reshape(n_grp_total, G).astype(jnp.int32)
    e = jnp.arange(n, dtype=jnp.int32)
    drow = ((e // S) * SUBPAD + e % S).reshape(n_grp_total, G)
    wpad = jnp.pad(weights, ((0, 0), (0, PADDED - EMB)))
    mesh = plsc.VectorSubcoreMesh(core_axis_name="c", subcore_axis_name="s")
    n_grp = n_grp_total // NUM_WORKERS
    out = pl.kernel(
        _body,
        out_type=jax.ShapeDtypeStruct((B * SUBPAD, PADDED), jnp.float32),
        mesh=mesh,
        scratch_types=[
            pltpu.VMEM((n_grp, G), jnp.int32),
            pltpu.VMEM((n_grp, G), jnp.int32),
            [pltpu.VMEM((G, PADDED), jnp.float32) for _ in range(NBUF)],
            [pltpu.SemaphoreType.DMA for _ in range(NBUF)],
            [pltpu.SemaphoreType.DMA for _ in range(NBUF)],
        ],
    )(idx, drow, wpad)
    out3 = out.reshape(B, SUBPAD, PADDED)[:, :S, :EMB]
    return with_layout_constraint(out3, Layout(major_to_minor=(0, 1, 2)))
